# Initial kernel scaffold; baseline (speedup 1.0000x reference)
import jax
import jax.numpy as jnp
from jax.experimental import pallas as pl


def kernel(q, v, embeddings):
    raise NotImplementedError("write your pallas kernel here")


# speedup vs baseline: 9.7897x; 9.7897x over previous
import functools

import jax
import jax.numpy as jnp
import numpy as np
from jax import lax
from jax.experimental import pallas as pl
from jax.experimental.pallas import tpu as pltpu
from jax.experimental.pallas import tpu_sc as plsc

NUM_BUCKETS = 32
MAX_DISTANCE = 128


def _table_body(q_len, T, emb_ref, out_ref):
    t = lax.broadcasted_iota(jnp.int32, (T, emb_ref.shape[1]), 0)
    n = (q_len - 1) - t
    num_buckets = NUM_BUCKETS // 2
    ret = jnp.where(n < 0, num_buckets, 0).astype(jnp.int32)
    n = jnp.abs(n)
    max_exact = num_buckets // 2
    is_small = n < max_exact
    val_if_large = max_exact + (
        jnp.log(n.astype(jnp.float32) / max_exact)
        / np.log(MAX_DISTANCE / max_exact)
        * (num_buckets - max_exact)
    ).astype(jnp.int32)
    val_if_large = jnp.minimum(val_if_large, num_buckets - 1)
    bucket = ret + jnp.where(is_small, n, val_if_large)

    acc = jnp.zeros((T, emb_ref.shape[1]), jnp.float32)
    for b in range(NUM_BUCKETS):
        acc = jnp.where(bucket == b, emb_ref[b, :][None, :], acc)
    out_ref[...] = acc


def _build_expanded(embeddings, q_len, v_len):
    T = q_len + v_len
    return pl.pallas_call(
        functools.partial(_table_body, q_len, T),
        out_shape=jax.ShapeDtypeStruct((T, embeddings.shape[1]), jnp.float32),
    )(embeddings)


def _sc_broadcast(expanded, q_len, v_len, dim):
    info = plsc.get_sparse_core_info()
    nw = info.num_cores * info.num_subcores
    rows_per_w = q_len // nw
    T = expanded.shape[0]

    @functools.partial(
        pl.kernel,
        out_type=jax.ShapeDtypeStruct((q_len, v_len, dim), jnp.float32),
        mesh=plsc.VectorSubcoreMesh(core_axis_name="c", subcore_axis_name="s"),
        scratch_types=[pltpu.VMEM((T, dim), jnp.float32)],
        compiler_params=pltpu.CompilerParams(use_tc_tiling_on_sc=False),
    )
    def body(exp_hbm, out_hbm, table_v):
        wid = lax.axis_index("s") * info.num_cores + lax.axis_index("c")
        pltpu.sync_copy(exp_hbm, table_v)
        base = wid * rows_per_w

        def row(r, carry):
            i = base + r
            start = (q_len - 1) - i
            pltpu.sync_copy(table_v.at[pl.ds(start, v_len), :], out_hbm.at[i])
            return carry

        lax.fori_loop(0, rows_per_w, row, 0)

    return body(expanded)


def kernel(q, v, embeddings):
    q_len = q.shape[1]
    v_len = v.shape[1]
    dim = embeddings.shape[1]
    expanded = _build_expanded(embeddings, q_len, v_len)
    return _sc_broadcast(expanded, q_len, v_len, dim)


# speedup vs baseline: 152.9742x; 15.6261x over previous
import functools

import jax
import jax.numpy as jnp
import numpy as np
from jax import lax
from jax.experimental import pallas as pl
from jax.experimental.pallas import tpu as pltpu

NUM_BUCKETS = 32
MAX_DISTANCE = 128


def _bucket_ids(q_len, shape, iota_dim):
    t = lax.broadcasted_iota(jnp.int32, shape, iota_dim)
    n = (q_len - 1) - t
    num_buckets = NUM_BUCKETS // 2
    ret = jnp.where(n < 0, num_buckets, 0).astype(jnp.int32)
    n = jnp.abs(n)
    max_exact = num_buckets // 2
    is_small = n < max_exact
    val_if_large = max_exact + (
        jnp.log(n.astype(jnp.float32) / max_exact)
        / np.log(MAX_DISTANCE / max_exact)
        * (num_buckets - max_exact)
    ).astype(jnp.int32)
    val_if_large = jnp.minimum(val_if_large, num_buckets - 1)
    return ret + jnp.where(is_small, n, val_if_large)


def _tc_body(q_len, v_len, dim, bi, tw, embT_ref, out_ref, table_ref):
    pid = pl.program_id(0)

    @pl.when(pid == 0)
    def _build_table():
        bucket = _bucket_ids(q_len, (dim, tw), 1)
        acc = jnp.zeros((dim, tw), jnp.float32)
        for b in range(NUM_BUCKETS):
            acc = jnp.where(bucket == b, embT_ref[:, b : b + 1], acc)
        table_ref[...] = acc

    win = v_len + 128
    for r in range(bi):
        i = pid * bi + r
        s = (q_len - 1) - i
        k128 = (s // 128) * 128
        phi = s - k128
        w = table_ref[:, pl.ds(pl.multiple_of(k128, 128), win)]
        rolled = pltpu.roll(w, jnp.where(phi == 0, 0, win - phi), 1)
        out_ref[r] = rolled[:, :v_len]


def _tc_call(embT, q_len, v_len, dim, interpret=False):
    bi = 32
    tw = -(-(q_len + v_len) // 128) * 128
    return pl.pallas_call(
        functools.partial(_tc_body, q_len, v_len, dim, bi, tw),
        grid=(q_len // bi,),
        in_specs=[pl.BlockSpec((dim, NUM_BUCKETS), lambda i: (0, 0))],
        out_specs=pl.BlockSpec((bi, dim, v_len), lambda i: (i, 0, 0)),
        out_shape=jax.ShapeDtypeStruct((q_len, dim, v_len), jnp.float32),
        scratch_shapes=[pltpu.VMEM((dim, tw), jnp.float32)],
        interpret=interpret,
    )(embT)


def kernel(q, v, embeddings):
    q_len = q.shape[1]
    v_len = v.shape[1]
    dim = embeddings.shape[1]
    res = _tc_call(embeddings.T, q_len, v_len, dim)
    return jnp.transpose(res, (0, 2, 1))
